# Initial kernel scaffold; baseline (speedup 1.0000x reference)
import jax
import jax.numpy as jnp
from jax.experimental import pallas as pl


def kernel(x, edge_index, W_lin1, b_lin1, W_agg1, b_agg1, W_lin2, b_lin2, W_agg2, b_agg2, W_post1, b_post1, W_post2, b_post2):
    raise NotImplementedError("write your pallas kernel here")


# speedup vs baseline: 6.0262x; 6.0262x over previous
import functools

import jax
import jax.numpy as jnp
from jax import lax
from jax.experimental import pallas as pl
from jax.experimental.pallas import tpu as pltpu
from jax.experimental.pallas import tpu_sc as plsc

_N, _E, _D, _H, _O = 10000, 320000, 128, 128, 64
_NC, _NS = 2, 16
_NW = _NC * _NS
_CW = 64
_CH = 160
_EPW = _CH * _CW
_EPAD = _NW * _EPW
_NPAD = 10112
_RPT = _NPAD // _NS
_CNT = _NPAD
_CPT = _CNT // _NS
_SZS = (128, 128, 128, 128, 120)
_SZA = (64,) * 9 + (56,)
_BLK = 2000


_MESH = plsc.VectorSubcoreMesh(core_axis_name="c", subcore_axis_name="s",
                               num_cores=_NC, num_subcores=_NS)


@functools.partial(
    pl.kernel, mesh=_MESH,
    out_type=jax.ShapeDtypeStruct((_NC, _NPAD, _H), jnp.float32),
    scratch_types=[
        pltpu.VMEM((40, _CW), jnp.int32),
        pltpu.VMEM((40, _CW), jnp.int32),
        pltpu.VMEM((_CW, _H), jnp.float32),
        pltpu.VMEM_SHARED((_NPAD, _H), jnp.float32),
    ])
def _segsum(y_hbm, srcw_hbm, dstw_hbm, part_hbm, src_v, dst_v, rows_v, acc_sh):
    cid = lax.axis_index("c")
    sid = lax.axis_index("s")
    wid = sid * _NC + cid

    zero16 = jnp.zeros((16,), jnp.float32)

    def _zrow(i, carry):
        for c in range(8):
            rows_v[i, pl.ds(c * 16, 16)] = zero16
        return carry

    lax.fori_loop(0, _CW, _zrow, 0)

    base = sid * _RPT
    off = 0
    for sz in _SZA:
        pltpu.sync_copy(rows_v.at[pl.ds(0, sz)],
                        acc_sh.at[pl.ds(base + off, sz)])
        off += sz
    plsc.subcore_barrier()

    def _chunk(j, carry):
        pltpu.sync_copy(y_hbm.at[src_v.at[j]], rows_v)
        pltpu.sync_copy(rows_v, acc_sh.at[dst_v.at[j]], add=True)
        return carry

    def _blk(h, carry):
        pltpu.sync_copy(srcw_hbm.at[wid, pl.ds(h * 40, 40)], src_v)
        pltpu.sync_copy(dstw_hbm.at[wid, pl.ds(h * 40, 40)], dst_v)
        lax.fori_loop(0, 40, _chunk, 0)
        return carry

    lax.fori_loop(0, _CH // 40, _blk, 0)
    plsc.subcore_barrier()

    off = 0
    for sz in _SZA:
        pltpu.sync_copy(acc_sh.at[pl.ds(base + off, sz)],
                        rows_v.at[pl.ds(0, sz)])
        pltpu.sync_copy(rows_v.at[pl.ds(0, sz)],
                        part_hbm.at[cid, pl.ds(base + off, sz)])
        off += sz


@functools.partial(
    pl.kernel, mesh=_MESH,
    out_type=jax.ShapeDtypeStruct((_NC, _CNT, 16), jnp.float32),
    scratch_types=[
        pltpu.VMEM((_CH, _CW), jnp.int32),
        pltpu.VMEM((_CW, 16), jnp.float32),
        pltpu.VMEM((128, 16), jnp.float32),
        pltpu.VMEM_SHARED((_CNT, 16), jnp.float32),
    ])
def _segcnt(dstw_hbm, cnt_hbm, dst_v, ones_v, zc_v, cnt_sh):
    cid = lax.axis_index("c")
    sid = lax.axis_index("s")
    wid = sid * _NC + cid

    zero16 = jnp.zeros((16,), jnp.float32)
    one16 = jnp.ones((16,), jnp.float32)

    def _fill(i, carry):
        zc_v[i, :] = zero16
        return carry

    def _fill_ones(i, carry):
        ones_v[i, :] = one16
        return carry

    lax.fori_loop(0, 128, _fill, 0)
    lax.fori_loop(0, _CW, _fill_ones, 0)

    cb = sid * _CPT
    off = 0
    for sz in _SZS:
        pltpu.sync_copy(zc_v.at[pl.ds(0, sz)],
                        cnt_sh.at[pl.ds(cb + off, sz)])
        off += sz
    plsc.subcore_barrier()

    pltpu.sync_copy(dstw_hbm.at[wid], dst_v)

    def _chunk(j, carry):
        pltpu.sync_copy(ones_v, cnt_sh.at[dst_v.at[j]], add=True)
        return carry

    lax.fori_loop(0, _CH, _chunk, 0)
    plsc.subcore_barrier()

    off = 0
    for sz in _SZS:
        pltpu.sync_copy(cnt_sh.at[pl.ds(cb + off, sz)],
                        zc_v.at[pl.ds(0, sz)])
        pltpu.sync_copy(zc_v.at[pl.ds(0, sz)],
                        cnt_hbm.at[cid, pl.ds(cb + off, sz)])
        off += sz


def _dot(a, b, preferred_element_type=jnp.float32):
    return jax.lax.dot(a, b, preferred_element_type=preferred_element_type)


def _lin_relu_body(x_ref, w_ref, b_ref, o_ref):
    o_ref[...] = jnp.maximum(
        _dot(x_ref[...], w_ref[...], preferred_element_type=jnp.float32)
        + b_ref[...], 0.0)


def _mid_body(x_ref, p0_ref, p1_ref, c0_ref, c1_ref, wa_ref, wb_ref, ba_ref,
              wl_ref, bl_ref, h_ref, y_ref):
    cnt = jnp.maximum(c0_ref[...] + c1_ref[...], 1.0)
    aggr = (p0_ref[...] + p1_ref[...]) / cnt
    pre = (_dot(x_ref[...], wa_ref[...], preferred_element_type=jnp.float32)
           + _dot(aggr, wb_ref[...], preferred_element_type=jnp.float32)
           + ba_ref[...])
    h = jnp.maximum(pre, 0.0)
    nrm = jnp.maximum(jnp.sqrt(jnp.sum(h * h, axis=1, keepdims=True)), 1e-12)
    h = h / nrm
    h_ref[...] = h
    y_ref[...] = jnp.maximum(
        _dot(h, wl_ref[...], preferred_element_type=jnp.float32)
        + bl_ref[...], 0.0)


def _out_body(h_ref, p0_ref, p1_ref, c0_ref, c1_ref, wa_ref, wb_ref, ba_ref,
              w1_ref, b1_ref, w2_ref, b2_ref, o_ref):
    cnt = jnp.maximum(c0_ref[...] + c1_ref[...], 1.0)
    aggr = (p0_ref[...] + p1_ref[...]) / cnt
    pre = (_dot(h_ref[...], wa_ref[...], preferred_element_type=jnp.float32)
           + _dot(aggr, wb_ref[...], preferred_element_type=jnp.float32)
           + ba_ref[...])
    h2 = jnp.maximum(pre, 0.0)
    nrm = jnp.maximum(jnp.sqrt(jnp.sum(h2 * h2, axis=1, keepdims=True)), 1e-12)
    h2 = h2 / nrm
    t = _dot(h2, w1_ref[...], preferred_element_type=jnp.float32) + b1_ref[...]
    o = _dot(t, w2_ref[...], preferred_element_type=jnp.float32) + b2_ref[...]
    m = jnp.max(o, axis=1, keepdims=True)
    lse = jnp.log(jnp.sum(jnp.exp(o - m), axis=1, keepdims=True)) + m
    o_ref[...] = o - lse


def _row_spec(cols):
    return pl.BlockSpec((_BLK, cols), lambda i: (i, 0))


def _full_spec(r, c):
    return pl.BlockSpec((r, c), lambda i: (0, 0))


_GRID = (_N // _BLK,)

_lin_relu = pl.pallas_call(
    _lin_relu_body,
    grid=_GRID,
    in_specs=[_row_spec(_D), _full_spec(_D, _H), _full_spec(1, _H)],
    out_specs=_row_spec(_H),
    out_shape=jax.ShapeDtypeStruct((_N, _H), jnp.float32),
)

_mid = pl.pallas_call(
    _mid_body,
    grid=_GRID,
    in_specs=[_row_spec(_D), _row_spec(_H), _row_spec(_H),
              _row_spec(1), _row_spec(1),
              _full_spec(_D, _H), _full_spec(_H, _H), _full_spec(1, _H),
              _full_spec(_H, _H), _full_spec(1, _H)],
    out_specs=[_row_spec(_H), _row_spec(_H)],
    out_shape=[jax.ShapeDtypeStruct((_N, _H), jnp.float32),
               jax.ShapeDtypeStruct((_N, _H), jnp.float32)],
)

_out = pl.pallas_call(
    _out_body,
    grid=_GRID,
    in_specs=[_row_spec(_H), _row_spec(_H), _row_spec(_H),
              _row_spec(1), _row_spec(1),
              _full_spec(_H, _H), _full_spec(_H, _H), _full_spec(1, _H),
              _full_spec(_H, _H), _full_spec(1, _H),
              _full_spec(_H, _O), _full_spec(1, _O)],
    out_specs=_row_spec(_O),
    out_shape=jax.ShapeDtypeStruct((_N, _O), jnp.float32),
)


def kernel(x, edge_index, W_lin1, b_lin1, W_agg1, b_agg1, W_lin2, b_lin2,
           W_agg2, b_agg2, W_post1, b_post1, W_post2, b_post2):
    src = edge_index[0]
    dst = edge_index[1]
    padk = jnp.arange(_EPAD - _E, dtype=jnp.int32)
    src_p = jnp.concatenate([src, (padk * 997) % _N]).reshape(_NW, _CH, _CW)
    dst_p = jnp.concatenate([dst, _N + padk % (_NPAD - _N)]).reshape(_NW, _CH, _CW)

    y1 = _lin_relu(x, W_lin1, b_lin1.reshape(1, _H))
    cnt1 = _segcnt(dst_p)
    part1 = _segsum(y1, src_p, dst_p)
    p0, p1 = part1[0, :_N], part1[1, :_N]
    c0 = cnt1[0, :_N, 0:1]
    c1 = cnt1[1, :_N, 0:1]
    h1, y2 = _mid(x, p0, p1, c0, c1, W_agg1[:_D], W_agg1[_D:],
                  b_agg1.reshape(1, _H), W_lin2, b_lin2.reshape(1, _H))
    part2 = _segsum(y2, src_p, dst_p)
    q0, q1 = part2[0, :_N], part2[1, :_N]
    return _out(h1, q0, q1, c0, c1, W_agg2[:_H], W_agg2[_H:],
                b_agg2.reshape(1, _H), W_post1, b_post1.reshape(1, _H),
                W_post2, b_post2.reshape(1, _O))


# speedup vs baseline: 7.5439x; 1.2519x over previous
import functools

import jax
import jax.numpy as jnp
from jax import lax
from jax.experimental import pallas as pl
from jax.experimental.pallas import tpu as pltpu
from jax.experimental.pallas import tpu_sc as plsc

_N, _E, _D, _H, _O = 10000, 320000, 128, 128, 64
_NC, _NS = 2, 16
_NW = _NC * _NS
_CW = 64
_CH = 160
_EPW = _CH * _CW
_EPAD = _NW * _EPW
_NPAD = 10112
_RPT = _NPAD // _NS
_CNT = _NPAD
_CPT = _CNT // _NS
_SZS = (128, 128, 128, 128, 120)
_SZA = (64,) * 9 + (56,)
_BLK = 2000


_MESH = plsc.VectorSubcoreMesh(core_axis_name="c", subcore_axis_name="s",
                               num_cores=_NC, num_subcores=_NS)


@functools.partial(
    pl.kernel, mesh=_MESH,
    out_type=jax.ShapeDtypeStruct((_NC, _NPAD, _H), jnp.float32),
    scratch_types=[
        pltpu.VMEM((40, _CW), jnp.int32),
        pltpu.VMEM((40, _CW), jnp.int32),
        pltpu.VMEM((_CW, _H), jnp.float32),
        pltpu.VMEM((_CW, _H), jnp.float32),
        pltpu.VMEM_SHARED((_NPAD, _H), jnp.float32),
        pltpu.SemaphoreType.DMA,
        pltpu.SemaphoreType.DMA,
    ])
def _segsum(y_hbm, srcw_hbm, dstw_hbm, part_hbm, src_v, dst_v, rows_v,
            rows2_v, acc_sh, sem_a, sem_b):
    cid = lax.axis_index("c")
    sid = lax.axis_index("s")
    wid = sid * _NC + cid

    zero16 = jnp.zeros((16,), jnp.float32)

    def _zrow(i, carry):
        for c in range(8):
            rows_v[i, pl.ds(c * 16, 16)] = zero16
        return carry

    lax.fori_loop(0, _CW, _zrow, 0)

    base = sid * _RPT
    off = 0
    for sz in _SZA:
        pltpu.sync_copy(rows_v.at[pl.ds(0, sz)],
                        acc_sh.at[pl.ds(base + off, sz)])
        off += sz
    plsc.subcore_barrier()

    def _pair(p, carry):
        j0 = 2 * p
        cp_a = pltpu.async_copy(y_hbm.at[src_v.at[j0]], rows_v, sem_a)
        cp_b = pltpu.async_copy(y_hbm.at[src_v.at[j0 + 1]], rows2_v, sem_b)
        cp_a.wait()
        pltpu.sync_copy(rows_v, acc_sh.at[dst_v.at[j0]], add=True)
        cp_b.wait()
        pltpu.sync_copy(rows2_v, acc_sh.at[dst_v.at[j0 + 1]], add=True)
        return carry

    def _blk(h, carry):
        pltpu.sync_copy(srcw_hbm.at[wid, pl.ds(h * 40, 40)], src_v)
        pltpu.sync_copy(dstw_hbm.at[wid, pl.ds(h * 40, 40)], dst_v)
        lax.fori_loop(0, 20, _pair, 0)
        return carry

    lax.fori_loop(0, _CH // 40, _blk, 0)
    plsc.subcore_barrier()

    off = 0
    for sz in _SZA:
        pltpu.sync_copy(acc_sh.at[pl.ds(base + off, sz)],
                        rows_v.at[pl.ds(0, sz)])
        pltpu.sync_copy(rows_v.at[pl.ds(0, sz)],
                        part_hbm.at[cid, pl.ds(base + off, sz)])
        off += sz


@functools.partial(
    pl.kernel, mesh=_MESH,
    out_type=jax.ShapeDtypeStruct((_NC, _CNT, 16), jnp.float32),
    scratch_types=[
        pltpu.VMEM((_CH, _CW), jnp.int32),
        pltpu.VMEM((_CW, 16), jnp.float32),
        pltpu.VMEM((128, 16), jnp.float32),
        pltpu.VMEM_SHARED((_CNT, 16), jnp.float32),
    ])
def _segcnt(dstw_hbm, cnt_hbm, dst_v, ones_v, zc_v, cnt_sh):
    cid = lax.axis_index("c")
    sid = lax.axis_index("s")
    wid = sid * _NC + cid

    zero16 = jnp.zeros((16,), jnp.float32)
    one16 = jnp.ones((16,), jnp.float32)

    def _fill(i, carry):
        zc_v[i, :] = zero16
        return carry

    def _fill_ones(i, carry):
        ones_v[i, :] = one16
        return carry

    lax.fori_loop(0, 128, _fill, 0)
    lax.fori_loop(0, _CW, _fill_ones, 0)

    cb = sid * _CPT
    off = 0
    for sz in _SZS:
        pltpu.sync_copy(zc_v.at[pl.ds(0, sz)],
                        cnt_sh.at[pl.ds(cb + off, sz)])
        off += sz
    plsc.subcore_barrier()

    pltpu.sync_copy(dstw_hbm.at[wid], dst_v)

    def _chunk(j, carry):
        pltpu.sync_copy(ones_v, cnt_sh.at[dst_v.at[j]], add=True)
        return carry

    lax.fori_loop(0, _CH, _chunk, 0)
    plsc.subcore_barrier()

    off = 0
    for sz in _SZS:
        pltpu.sync_copy(cnt_sh.at[pl.ds(cb + off, sz)],
                        zc_v.at[pl.ds(0, sz)])
        pltpu.sync_copy(zc_v.at[pl.ds(0, sz)],
                        cnt_hbm.at[cid, pl.ds(cb + off, sz)])
        off += sz


def _dot(a, b, preferred_element_type=jnp.float32):
    return jax.lax.dot(a, b, preferred_element_type=preferred_element_type)


def _lin_relu_body(x_ref, w_ref, b_ref, o_ref):
    o_ref[...] = jnp.maximum(
        _dot(x_ref[...], w_ref[...], preferred_element_type=jnp.float32)
        + b_ref[...], 0.0)


def _mid_body(x_ref, p0_ref, p1_ref, c0_ref, c1_ref, wa_ref, wb_ref, ba_ref,
              wl_ref, bl_ref, h_ref, y_ref):
    cnt = jnp.maximum(c0_ref[...] + c1_ref[...], 1.0)
    aggr = (p0_ref[...] + p1_ref[...]) / cnt
    pre = (_dot(x_ref[...], wa_ref[...], preferred_element_type=jnp.float32)
           + _dot(aggr, wb_ref[...], preferred_element_type=jnp.float32)
           + ba_ref[...])
    h = jnp.maximum(pre, 0.0)
    nrm = jnp.maximum(jnp.sqrt(jnp.sum(h * h, axis=1, keepdims=True)), 1e-12)
    h = h / nrm
    h_ref[...] = h
    y_ref[...] = jnp.maximum(
        _dot(h, wl_ref[...], preferred_element_type=jnp.float32)
        + bl_ref[...], 0.0)


def _out_body(h_ref, p0_ref, p1_ref, c0_ref, c1_ref, wa_ref, wb_ref, ba_ref,
              w1_ref, b1_ref, w2_ref, b2_ref, o_ref):
    cnt = jnp.maximum(c0_ref[...] + c1_ref[...], 1.0)
    aggr = (p0_ref[...] + p1_ref[...]) / cnt
    pre = (_dot(h_ref[...], wa_ref[...], preferred_element_type=jnp.float32)
           + _dot(aggr, wb_ref[...], preferred_element_type=jnp.float32)
           + ba_ref[...])
    h2 = jnp.maximum(pre, 0.0)
    nrm = jnp.maximum(jnp.sqrt(jnp.sum(h2 * h2, axis=1, keepdims=True)), 1e-12)
    h2 = h2 / nrm
    t = _dot(h2, w1_ref[...], preferred_element_type=jnp.float32) + b1_ref[...]
    o = _dot(t, w2_ref[...], preferred_element_type=jnp.float32) + b2_ref[...]
    m = jnp.max(o, axis=1, keepdims=True)
    lse = jnp.log(jnp.sum(jnp.exp(o - m), axis=1, keepdims=True)) + m
    o_ref[...] = o - lse


def _row_spec(cols):
    return pl.BlockSpec((_BLK, cols), lambda i: (i, 0))


def _full_spec(r, c):
    return pl.BlockSpec((r, c), lambda i: (0, 0))


_GRID = (_N // _BLK,)

_lin_relu = pl.pallas_call(
    _lin_relu_body,
    grid=_GRID,
    in_specs=[_row_spec(_D), _full_spec(_D, _H), _full_spec(1, _H)],
    out_specs=_row_spec(_H),
    out_shape=jax.ShapeDtypeStruct((_N, _H), jnp.float32),
)

_mid = pl.pallas_call(
    _mid_body,
    grid=_GRID,
    in_specs=[_row_spec(_D), _row_spec(_H), _row_spec(_H),
              _row_spec(1), _row_spec(1),
              _full_spec(_D, _H), _full_spec(_H, _H), _full_spec(1, _H),
              _full_spec(_H, _H), _full_spec(1, _H)],
    out_specs=[_row_spec(_H), _row_spec(_H)],
    out_shape=[jax.ShapeDtypeStruct((_N, _H), jnp.float32),
               jax.ShapeDtypeStruct((_N, _H), jnp.float32)],
)

_out = pl.pallas_call(
    _out_body,
    grid=_GRID,
    in_specs=[_row_spec(_H), _row_spec(_H), _row_spec(_H),
              _row_spec(1), _row_spec(1),
              _full_spec(_H, _H), _full_spec(_H, _H), _full_spec(1, _H),
              _full_spec(_H, _H), _full_spec(1, _H),
              _full_spec(_H, _O), _full_spec(1, _O)],
    out_specs=_row_spec(_O),
    out_shape=jax.ShapeDtypeStruct((_N, _O), jnp.float32),
)


def kernel(x, edge_index, W_lin1, b_lin1, W_agg1, b_agg1, W_lin2, b_lin2,
           W_agg2, b_agg2, W_post1, b_post1, W_post2, b_post2):
    src = edge_index[0]
    dst = edge_index[1]
    padk = jnp.arange(_EPAD - _E, dtype=jnp.int32)
    src_p = jnp.concatenate([src, (padk * 997) % _N]).reshape(_NW, _CH, _CW)
    dst_p = jnp.concatenate([dst, _N + padk % (_NPAD - _N)]).reshape(_NW, _CH, _CW)

    y1 = _lin_relu(x, W_lin1, b_lin1.reshape(1, _H))
    cnt1 = _segcnt(dst_p)
    part1 = _segsum(y1, src_p, dst_p)
    p0, p1 = part1[0, :_N], part1[1, :_N]
    c0 = cnt1[0, :_N, 0:1]
    c1 = cnt1[1, :_N, 0:1]
    h1, y2 = _mid(x, p0, p1, c0, c1, W_agg1[:_D], W_agg1[_D:],
                  b_agg1.reshape(1, _H), W_lin2, b_lin2.reshape(1, _H))
    part2 = _segsum(y2, src_p, dst_p)
    q0, q1 = part2[0, :_N], part2[1, :_N]
    return _out(h1, q0, q1, c0, c1, W_agg2[:_H], W_agg2[_H:],
                b_agg2.reshape(1, _H), W_post1, b_post1.reshape(1, _H),
                W_post2, b_post2.reshape(1, _O))
